# baseline (speedup 1.0000x reference)
import functools

import jax
import jax.numpy as jnp
from jax import lax
from jax.experimental import pallas as pl
from jax.experimental.pallas import tpu as pltpu
from jax.experimental.pallas import tpu_sc as plsc

C_LEN = 1_000_000
U_LEN = 100_000
EMBED = 64
BATCH = 16384

NUM_CORES = 2
NUM_SUBCORES = 16
NW = NUM_CORES * NUM_SUBCORES
BPW = BATCH // NW
CHUNK = 128
NCH = BPW // CHUNK
LANES = 16

_mesh = plsc.VectorSubcoreMesh(core_axis_name="c", subcore_axis_name="s")


@functools.partial(
    pl.kernel,
    mesh=_mesh,
    out_type=jax.ShapeDtypeStruct((BATCH,), jnp.float32),
    compiler_params=pltpu.CompilerParams(
        needs_layout_passes=False, use_tc_tiling_on_sc=False),
    scratch_types=[
        pltpu.VMEM((NCH, CHUNK), jnp.int32),
        pltpu.VMEM((NCH, CHUNK), jnp.int32),
        pltpu.VMEM((BPW, EMBED), jnp.float32),
        pltpu.VMEM((BPW, EMBED), jnp.float32),
        pltpu.VMEM((BPW,), jnp.float32),
        pltpu.SemaphoreType.DMA,
    ],
)
def _gmf_sc(c_idx_hbm, u_idx_hbm, c_tab_hbm, u_tab_hbm, out_hbm,
            cidx_v, uidx_v, crows_v, urows_v, out_v, sem):
    wid = lax.axis_index("s") * NUM_CORES + lax.axis_index("c")
    base = wid * BPW
    row_base = wid * NCH

    pltpu.sync_copy(c_idx_hbm.at[pl.ds(row_base, NCH)], cidx_v)
    pltpu.sync_copy(u_idx_hbm.at[pl.ds(row_base, NCH)], uidx_v)

    copies = []
    for j in range(NCH):
        copies.append(pltpu.async_copy(
            c_tab_hbm.at[cidx_v.at[j]],
            crows_v.at[pl.ds(j * CHUNK, CHUNK)], sem))
        copies.append(pltpu.async_copy(
            u_tab_hbm.at[uidx_v.at[j]],
            urows_v.at[pl.ds(j * CHUNK, CHUNK)], sem))
    for c in copies:
        c.wait()

    def body(g, carry):
        rows = g * LANES + lax.iota(jnp.int32, LANES)
        accs = [None] * 4
        for d in range(EMBED):
            cols = jnp.full((LANES,), d, jnp.int32)
            cv = plsc.load_gather(crows_v, [rows, cols])
            uv = plsc.load_gather(urows_v, [rows, cols])
            p = cv * uv
            k = d % 4
            accs[k] = p if accs[k] is None else accs[k] + p
        out_v[pl.ds(g * LANES, LANES)] = (accs[0] + accs[1]) + (accs[2] + accs[3])
        return carry

    lax.fori_loop(0, BPW // LANES, body, 0)

    pltpu.sync_copy(out_v, out_hbm.at[pl.ds(base, BPW)])


def kernel(c_idx, u_idx, c_table, u_table):
    c_idx2 = jnp.asarray(c_idx, jnp.int32).reshape(BATCH // CHUNK, CHUNK)
    u_idx2 = jnp.asarray(u_idx, jnp.int32).reshape(BATCH // CHUNK, CHUNK)
    out = _gmf_sc(c_idx2, u_idx2, c_table, u_table)
    return out.reshape(BATCH, 1)
